# baseline (speedup 1.0000x reference)
import jax
import jax.numpy as jnp
from jax import lax
from jax.experimental import pallas as pl
from jax.experimental.pallas import tpu as pltpu
from jax.experimental.pallas import tpu_sc as plsc

_S = 30.0
_M = 0.35

_B = 1024
_C = 100000
_CLS_BLOCK = 1000
_NW = 16
_RPW = _B // _NW


def _tc_body(x_ref, o_ref):
    o_ref[...] = jnp.clip(x_ref[...], -1.0, 1.0) * _S


def _tc_clip_scale(lt):
    c, b = lt.shape
    return pl.pallas_call(
        _tc_body,
        grid=(c // _CLS_BLOCK,),
        in_specs=[pl.BlockSpec((_CLS_BLOCK, b), lambda j: (j, 0))],
        out_specs=pl.BlockSpec((_CLS_BLOCK, b), lambda j: (j, 0)),
        out_shape=jax.ShapeDtypeStruct((c, b), jnp.float32),
    )(lt)


def _sc_margin_body(out_hbm, lab_hbm, lab_all, lab_mine, rows_v, sem):
    wid = lax.axis_index("s")
    base = wid * _RPW
    pltpu.sync_copy(lab_hbm, lab_all)
    pltpu.sync_copy(lab_hbm.at[pl.ds(base, _RPW)], lab_mine)
    pltpu.async_copy(out_hbm.at[lab_mine], rows_v, sem).wait()

    splat_dnums = lax.GatherDimensionNumbers(
        offset_dims=(), collapsed_slice_dims=(0,), start_index_map=(0,)
    )
    for g in range(_RPW // 16):
        vg = lab_mine[pl.ds(g * 16, 16)]
        for i in range(16):
            k = g * 16 + i
            idx16 = jnp.full((16, 1), i, jnp.int32)
            lk = lax.gather(
                vg,
                idx16,
                splat_dnums,
                slice_sizes=(1,),
                mode=lax.GatherScatterMode.PROMISE_IN_BOUNDS,
            )

            @pl.loop(0, _B // 16)
            def _per_chunk(c, k=k, lk=lk):
                sl = pl.ds(c * 16, 16)
                m = lab_all[sl] == lk
                rows_v[k, sl] = rows_v[k, sl] - jnp.where(m, _M * _S, 0.0)

    plsc.subcore_barrier()
    pltpu.async_copy(rows_v, out_hbm.at[lab_mine], sem).wait()


_sc_margin = pl.kernel(
    _sc_margin_body,
    mesh=plsc.VectorSubcoreMesh(
        core_axis_name="c", subcore_axis_name="s", num_cores=1
    ),
    scratch_types=[
        pltpu.VMEM((_B,), jnp.int32),
        pltpu.VMEM((_RPW,), jnp.int32),
        pltpu.VMEM((_RPW, _B), jnp.float32),
        pltpu.SemaphoreType.DMA,
    ],
)


def kernel(logits, labels):
    b, c = logits.shape
    lt = jnp.swapaxes(logits, 0, 1)
    outt = _tc_clip_scale(lt)
    ref = jax.new_ref(outt)
    _sc_margin(ref, labels.reshape(b))
    return jnp.swapaxes(jax.freeze(ref), 0, 1)


# speedup vs baseline: 1.0347x; 1.0347x over previous
import jax
import jax.numpy as jnp
from jax import lax
from jax.experimental import pallas as pl
from jax.experimental.pallas import tpu as pltpu
from jax.experimental.pallas import tpu_sc as plsc

_S = 30.0
_M = 0.35

_B = 1024
_C = 100000
_CLS_BLOCK = 1000
_NW = 32
_RPW = _B // _NW


def _tc_body(x_ref, o_ref):
    o_ref[...] = jnp.clip(x_ref[...], -1.0, 1.0) * _S


def _tc_clip_scale(lt):
    c, b = lt.shape
    return pl.pallas_call(
        _tc_body,
        grid=(c // _CLS_BLOCK,),
        in_specs=[pl.BlockSpec((_CLS_BLOCK, b), lambda j: (j, 0))],
        out_specs=pl.BlockSpec((_CLS_BLOCK, b), lambda j: (j, 0)),
        out_shape=jax.ShapeDtypeStruct((c, b), jnp.float32),
    )(lt)


def _sc_margin_body(out_hbm, x_hbm, lab_hbm, lab_all, lab_mine, rows_v, sem):
    wid = lax.axis_index("s") * 2 + lax.axis_index("c")
    base = wid * _RPW
    pltpu.sync_copy(lab_hbm, lab_all)
    pltpu.sync_copy(lab_hbm.at[pl.ds(base, _RPW)], lab_mine)
    pltpu.async_copy(x_hbm.at[lab_mine], rows_v, sem).wait()

    splat_dnums = lax.GatherDimensionNumbers(
        offset_dims=(), collapsed_slice_dims=(0,), start_index_map=(0,)
    )
    for g in range(_RPW // 16):
        vg = lab_mine[pl.ds(g * 16, 16)]
        for i in range(16):
            k = g * 16 + i
            idx16 = jnp.full((16, 1), i, jnp.int32)
            lk = lax.gather(
                vg,
                idx16,
                splat_dnums,
                slice_sizes=(1,),
                mode=lax.GatherScatterMode.PROMISE_IN_BOUNDS,
            )

            @pl.loop(0, _B // 16, unroll=8)
            def _per_chunk(c, k=k, lk=lk):
                sl = pl.ds(c * 16, 16)
                v = jnp.clip(rows_v[k, sl], -1.0, 1.0) * _S
                m = lab_all[sl] == lk
                rows_v[k, sl] = v - jnp.where(m, _M * _S, 0.0)

    pltpu.async_copy(rows_v, out_hbm.at[lab_mine], sem).wait()


_sc_margin = pl.kernel(
    _sc_margin_body,
    mesh=plsc.VectorSubcoreMesh(core_axis_name="c", subcore_axis_name="s"),
    scratch_types=[
        pltpu.VMEM((_B,), jnp.int32),
        pltpu.VMEM((_RPW,), jnp.int32),
        pltpu.VMEM((_RPW, _B), jnp.float32),
        pltpu.SemaphoreType.DMA,
    ],
)


def kernel(logits, labels):
    b, c = logits.shape
    lt = jnp.swapaxes(logits, 0, 1)
    outt = _tc_clip_scale(lt)
    ref = jax.new_ref(outt)
    _sc_margin(ref, lt, labels.reshape(b))
    return jnp.swapaxes(jax.freeze(ref), 0, 1)


# speedup vs baseline: 1.0849x; 1.0485x over previous
import jax
import jax.numpy as jnp
from jax import lax
from jax.experimental import pallas as pl
from jax.experimental.pallas import tpu as pltpu
from jax.experimental.pallas import tpu_sc as plsc

_S = 30.0
_M = 0.35

_B = 1024
_C = 100000
_CLS_BLOCK = 1000
_NW = 32
_RPW = _B // _NW


def _tc_body(x_ref, o_ref):
    o_ref[...] = jnp.clip(x_ref[...], -1.0, 1.0) * _S


def _tc_clip_scale(lt):
    c, b = lt.shape
    return pl.pallas_call(
        _tc_body,
        grid=(c // _CLS_BLOCK,),
        in_specs=[pl.BlockSpec((_CLS_BLOCK, b), lambda j: (j, 0))],
        out_specs=pl.BlockSpec((_CLS_BLOCK, b), lambda j: (j, 0)),
        out_shape=jax.ShapeDtypeStruct((c, b), jnp.float32),
    )(lt)


def _sc_rows_body(x_hbm, lab_hbm, rows_hbm, lab_all, lab_mine, rows_v, sem):
    wid = lax.axis_index("s") * 2 + lax.axis_index("c")
    base = wid * _RPW
    pltpu.sync_copy(lab_hbm, lab_all)
    pltpu.sync_copy(lab_hbm.at[pl.ds(base, _RPW)], lab_mine)
    pltpu.async_copy(x_hbm.at[lab_mine], rows_v, sem).wait()

    splat_dnums = lax.GatherDimensionNumbers(
        offset_dims=(), collapsed_slice_dims=(0,), start_index_map=(0,)
    )
    for g in range(_RPW // 16):
        vg = lab_mine[pl.ds(g * 16, 16)]
        for i in range(16):
            k = g * 16 + i
            idx16 = jnp.full((16, 1), i, jnp.int32)
            lk = lax.gather(
                vg,
                idx16,
                splat_dnums,
                slice_sizes=(1,),
                mode=lax.GatherScatterMode.PROMISE_IN_BOUNDS,
            )

            @pl.loop(0, _B // 16, unroll=8)
            def _per_chunk(c, k=k, lk=lk):
                sl = pl.ds(c * 16, 16)
                v = jnp.clip(rows_v[k, sl], -1.0, 1.0) * _S
                m = lab_all[sl] == lk
                rows_v[k, sl] = v - jnp.where(m, _M * _S, 0.0)

    pltpu.sync_copy(rows_v, rows_hbm.at[pl.ds(base, _RPW)])


_sc_rows = pl.kernel(
    _sc_rows_body,
    out_type=jax.ShapeDtypeStruct((_B, _B), jnp.float32),
    mesh=plsc.VectorSubcoreMesh(core_axis_name="c", subcore_axis_name="s"),
    scratch_types=[
        pltpu.VMEM((_B,), jnp.int32),
        pltpu.VMEM((_RPW,), jnp.int32),
        pltpu.VMEM((_RPW, _B), jnp.float32),
        pltpu.SemaphoreType.DMA,
    ],
)


def _sc_scatter_body(out_hbm, rows_hbm, lab_hbm, lab_mine, rows_v, sem):
    wid = lax.axis_index("s") * 2 + lax.axis_index("c")
    base = wid * _RPW
    pltpu.sync_copy(lab_hbm.at[pl.ds(base, _RPW)], lab_mine)
    pltpu.sync_copy(rows_hbm.at[pl.ds(base, _RPW)], rows_v)
    pltpu.async_copy(rows_v, out_hbm.at[lab_mine], sem).wait()


_sc_scatter = pl.kernel(
    _sc_scatter_body,
    mesh=plsc.VectorSubcoreMesh(core_axis_name="c", subcore_axis_name="s"),
    scratch_types=[
        pltpu.VMEM((_RPW,), jnp.int32),
        pltpu.VMEM((_RPW, _B), jnp.float32),
        pltpu.SemaphoreType.DMA,
    ],
)


def kernel(logits, labels):
    b, c = logits.shape
    lt = jnp.swapaxes(logits, 0, 1)
    lab = labels.reshape(b)
    rows = _sc_rows(lt, lab)
    outt = _tc_clip_scale(lt)
    ref = jax.new_ref(outt)
    _sc_scatter(ref, rows, lab)
    return jnp.swapaxes(jax.freeze(ref), 0, 1)


# speedup vs baseline: 1.0958x; 1.0100x over previous
import jax
import jax.numpy as jnp
from jax import lax
from jax.experimental import pallas as pl
from jax.experimental.pallas import tpu as pltpu
from jax.experimental.pallas import tpu_sc as plsc

_S = 30.0
_M = 0.35

_B = 1024
_C = 100000
_CLS_BLOCK = 2000
_NW = 32
_RPW = _B // _NW


def _tc_body(x_ref, o_ref):
    o_ref[...] = jnp.clip(x_ref[...], -1.0, 1.0) * _S


def _tc_clip_scale(lt):
    c, b = lt.shape
    return pl.pallas_call(
        _tc_body,
        grid=(c // _CLS_BLOCK,),
        in_specs=[pl.BlockSpec((_CLS_BLOCK, b), lambda j: (j, 0))],
        out_specs=pl.BlockSpec((_CLS_BLOCK, b), lambda j: (j, 0)),
        out_shape=jax.ShapeDtypeStruct((c, b), jnp.float32),
    )(lt)


def _sc_rows_body(x_hbm, lab_hbm, rows_hbm, lab_all, lab_mine, rows_v, sem):
    wid = lax.axis_index("s") * 2 + lax.axis_index("c")
    base = wid * _RPW
    pltpu.sync_copy(lab_hbm, lab_all)
    pltpu.sync_copy(lab_hbm.at[pl.ds(base, _RPW)], lab_mine)
    pltpu.async_copy(x_hbm.at[lab_mine], rows_v, sem).wait()

    splat_dnums = lax.GatherDimensionNumbers(
        offset_dims=(), collapsed_slice_dims=(0,), start_index_map=(0,)
    )
    for g in range(_RPW // 16):
        vg = lab_mine[pl.ds(g * 16, 16)]
        for i in range(16):
            k = g * 16 + i
            idx16 = jnp.full((16, 1), i, jnp.int32)
            lk = lax.gather(
                vg,
                idx16,
                splat_dnums,
                slice_sizes=(1,),
                mode=lax.GatherScatterMode.PROMISE_IN_BOUNDS,
            )

            @pl.loop(0, _B // 16, unroll=8)
            def _per_chunk(c, k=k, lk=lk):
                sl = pl.ds(c * 16, 16)
                v = jnp.clip(rows_v[k, sl], -1.0, 1.0) * _S
                m = lab_all[sl] == lk
                rows_v[k, sl] = v - jnp.where(m, _M * _S, 0.0)

    pltpu.sync_copy(rows_v, rows_hbm.at[pl.ds(base, _RPW)])


_sc_rows = pl.kernel(
    _sc_rows_body,
    out_type=jax.ShapeDtypeStruct((_B, _B), jnp.float32),
    mesh=plsc.VectorSubcoreMesh(core_axis_name="c", subcore_axis_name="s"),
    scratch_types=[
        pltpu.VMEM((_B,), jnp.int32),
        pltpu.VMEM((_RPW,), jnp.int32),
        pltpu.VMEM((_RPW, _B), jnp.float32),
        pltpu.SemaphoreType.DMA,
    ],
)


def _sc_scatter_body(out_hbm, rows_hbm, lab_hbm, lab_mine, rows_v, sem):
    wid = lax.axis_index("s") * 2 + lax.axis_index("c")
    base = wid * _RPW
    pltpu.sync_copy(lab_hbm.at[pl.ds(base, _RPW)], lab_mine)
    pltpu.sync_copy(rows_hbm.at[pl.ds(base, _RPW)], rows_v)
    pltpu.async_copy(rows_v, out_hbm.at[lab_mine], sem).wait()


_sc_scatter = pl.kernel(
    _sc_scatter_body,
    mesh=plsc.VectorSubcoreMesh(core_axis_name="c", subcore_axis_name="s"),
    scratch_types=[
        pltpu.VMEM((_RPW,), jnp.int32),
        pltpu.VMEM((_RPW, _B), jnp.float32),
        pltpu.SemaphoreType.DMA,
    ],
)


def kernel(logits, labels):
    b, c = logits.shape
    lt = jnp.swapaxes(logits, 0, 1)
    lab = labels.reshape(b)
    rows = _sc_rows(lt, lab)
    outt = _tc_clip_scale(lt)
    ref = jax.new_ref(outt)
    _sc_scatter(ref, rows, lab)
    return jnp.swapaxes(jax.freeze(ref), 0, 1)
